# Initial kernel scaffold; baseline (speedup 1.0000x reference)
import functools

import jax
import jax.numpy as jnp
from jax import lax
from jax.experimental import pallas as pl
from jax.experimental.pallas import tpu as pltpu
from jax.experimental.pallas import tpu_sc as plsc

_V = 49408
_D = 768
_T = 77
_B = 1024
_NW = 32
_ROWS = _B * _T
_RPW = _ROWS // _NW
_CHUNK = 16
_NCH = _RPW // _CHUNK
_NBUF = 7
_NBLK = _NCH // _NBUF
_LANES = 16
_DV = _D // _LANES

_mesh = plsc.VectorSubcoreMesh(core_axis_name="c", subcore_axis_name="s")


@functools.partial(
    pl.kernel,
    out_type=jax.ShapeDtypeStruct((_B, _T, _D), jnp.float32),
    mesh=_mesh,
    scratch_types=(
        [pltpu.VMEM((_RPW,), jnp.int32),
         pltpu.VMEM((4, _D), jnp.float32)]
        + [pltpu.VMEM((_CHUNK, _D), jnp.float32) for _ in range(_NBUF)]
        + [pltpu.SemaphoreType.DMA for _ in range(2 * _NBUF)]
    ),
)
def _embed_sc(tok_ref, pos_ref, tab_ref, out_ref, idx_v, pos_v, *rest):
    bufs = rest[:_NBUF]
    gsems = rest[_NBUF:2 * _NBUF]
    ssems = rest[2 * _NBUF:3 * _NBUF]

    wid = lax.axis_index("s") * 2 + lax.axis_index("c")
    base = wid * _RPW
    t0 = base // _B

    pltpu.sync_copy(tok_ref.at[pl.ds(base, _RPW)], idx_v)
    pltpu.sync_copy(pos_ref.at[pl.ds(t0, 4)], pos_v)

    def fire_gather(k, b):
        pltpu.async_copy(
            tab_ref.at[idx_v.at[pl.ds(k * _CHUNK, _CHUNK)]], bufs[b], gsems[b])

    def wait_gather(k, b):
        pltpu.make_async_copy(
            tab_ref.at[idx_v.at[pl.ds(k * _CHUNK, _CHUNK)]], bufs[b],
            gsems[b]).wait()

    def out_slice(g):
        t = g // _B
        b0 = g % _B
        return out_ref.at[pl.ds(b0, _CHUNK), t]

    def wait_store(g, b):
        pltpu.make_async_copy(bufs[b], out_slice(g), ssems[b]).wait()

    def add_pos(buf, ti):
        def dv_body(dv, carry):
            off = dv * _LANES
            pv = pos_v[ti, pl.ds(off, _LANES)]
            for r in range(_CHUNK):
                buf[r, pl.ds(off, _LANES)] = buf[r, pl.ds(off, _LANES)] + pv
            return carry
        lax.fori_loop(0, _DV, dv_body, 0)

    for b in range(_NBUF):
        fire_gather(b, b)

    def block(o, carry):
        k0 = o * _NBUF
        for b in range(_NBUF):
            k = k0 + b
            g = base + k * _CHUNK
            ti = g // _B - t0
            wait_gather(k, b)
            add_pos(bufs[b], ti)
            pltpu.async_copy(bufs[b], out_slice(g), ssems[b])

        @pl.when(o < _NBLK - 1)
        def _refill():
            for b in range(_NBUF):
                k = k0 + b
                g = base + k * _CHUNK
                wait_store(g, b)
                fire_gather(k + _NBUF, b)

        return carry

    lax.fori_loop(0, _NBLK, block, 0)

    k0 = (_NBLK - 1) * _NBUF
    for b in range(_NBUF):
        g = base + (k0 + b) * _CHUNK
        wait_store(g, b)


def kernel(tokens, token_embd, position_embd):
    tokens_t = tokens.astype(jnp.int32).T.reshape(-1)
    pos_pad = jnp.pad(position_embd, ((0, 4 - _T % 4), (0, 0)))
    return _embed_sc(tokens_t, pos_pad, token_embd)


# speedup vs baseline: 1.1401x; 1.1401x over previous
import functools

import jax
import jax.numpy as jnp
from jax import lax
from jax.experimental import pallas as pl
from jax.experimental.pallas import tpu as pltpu
from jax.experimental.pallas import tpu_sc as plsc

_V = 49408
_D = 768
_T = 77
_B = 1024
_NW = 32
_ROWS = _B * _T
_RPW = _ROWS // _NW
_CHUNK = 16
_NCH = _RPW // _CHUNK
_NBUF = 7
_NBLK = _NCH // _NBUF
_LANES = 16
_DV = _D // _LANES

_mesh = plsc.VectorSubcoreMesh(core_axis_name="c", subcore_axis_name="s")


@functools.partial(
    pl.kernel,
    out_type=jax.ShapeDtypeStruct((_ROWS, _D), jnp.float32),
    mesh=_mesh,
    scratch_types=(
        [pltpu.VMEM((_RPW,), jnp.int32),
         pltpu.VMEM((4 * _D,), jnp.float32)]
        + [pltpu.VMEM((_CHUNK, _D), jnp.float32) for _ in range(_NBUF)]
        + [pltpu.VMEM((_CHUNK,), jnp.int32) for _ in range(_NBUF)]
        + [pltpu.SemaphoreType.DMA for _ in range(2 * _NBUF)]
    ),
)
def _embed_sc(tok_ref, pos_ref, tab_ref, out_ref, idx_v, pos_v, *rest):
    bufs = rest[:_NBUF]
    oidx = rest[_NBUF:2 * _NBUF]
    gsems = rest[2 * _NBUF:3 * _NBUF]
    ssems = rest[3 * _NBUF:4 * _NBUF]

    wid = lax.axis_index("s") * 2 + lax.axis_index("c")
    base = wid * _RPW
    t0 = base // _B

    pltpu.sync_copy(tok_ref.at[pl.ds(base, _RPW)], idx_v)
    pltpu.sync_copy(pos_ref.at[pl.ds(t0 * _D, 4 * _D)], pos_v)

    def fire_gather(k, b):
        pltpu.async_copy(
            tab_ref.at[idx_v.at[pl.ds(k * _CHUNK, _CHUNK)]], bufs[b], gsems[b])

    def wait_gather(k, b):
        pltpu.make_async_copy(
            tab_ref.at[idx_v.at[pl.ds(k * _CHUNK, _CHUNK)]], bufs[b],
            gsems[b]).wait()

    def wait_store(b):
        pltpu.make_async_copy(bufs[b], out_ref.at[oidx[b]], ssems[b]).wait()

    def add_pos(buf, ti):
        def dv_body(dv, carry):
            off = ti * _D + dv * _LANES
            pv = pos_v[pl.ds(off, _LANES)]
            for r in range(_CHUNK):
                buf[r, pl.ds(dv * _LANES, _LANES)] = (
                    buf[r, pl.ds(dv * _LANES, _LANES)] + pv)
            return carry
        lax.fori_loop(0, _DV, dv_body, 0)

    for b in range(_NBUF):
        fire_gather(b, b)

    def block(o, carry):
        k0 = o * _NBUF
        for b in range(_NBUF):
            k = k0 + b
            g = base + k * _CHUNK
            t = g // _B
            b0 = g % _B
            ti = t - t0
            oidx[b][...] = lax.iota(jnp.int32, _CHUNK) * _T + (b0 * _T + t)
            wait_gather(k, b)
            add_pos(bufs[b], ti)
            pltpu.async_copy(bufs[b], out_ref.at[oidx[b]], ssems[b])

        @pl.when(o < _NBLK - 1)
        def _refill():
            for b in range(_NBUF):
                wait_store(b)
                fire_gather(k0 + b + _NBUF, b)

        return carry

    lax.fori_loop(0, _NBLK, block, 0)

    for b in range(_NBUF):
        wait_store(b)


def kernel(tokens, token_embd, position_embd):
    tokens_t = tokens.astype(jnp.int32).T.reshape(-1)
    pos_flat = jnp.pad(position_embd, ((0, 3), (0, 0))).reshape(-1)
    out = _embed_sc(tokens_t, pos_flat, token_embd)
    return out.reshape(_B, _T, _D)


# speedup vs baseline: 1.2859x; 1.1278x over previous
import functools

import jax
import jax.numpy as jnp
from jax import lax
from jax.experimental import pallas as pl
from jax.experimental.pallas import tpu as pltpu
from jax.experimental.pallas import tpu_sc as plsc

_V = 49408
_D = 768
_T = 77
_B = 1024
_NW = 32
_ROWS = _B * _T
_RPW = _ROWS // _NW
_CHUNK = 16
_NCH = _RPW // _CHUNK
_NBUF = 7
_NBLK = _NCH // _NBUF
_LANES = 16
_DV = _D // _LANES

_mesh = plsc.VectorSubcoreMesh(core_axis_name="c", subcore_axis_name="s")


@functools.partial(
    pl.kernel,
    out_type=jax.ShapeDtypeStruct((_ROWS, _D), jnp.float32),
    mesh=_mesh,
    scratch_types=(
        [pltpu.VMEM((_RPW,), jnp.int32),
         pltpu.VMEM((4 * _D,), jnp.float32)]
        + [pltpu.VMEM((_CHUNK, _D), jnp.float32) for _ in range(_NBUF)]
        + [pltpu.VMEM((_CHUNK,), jnp.int32) for _ in range(_NBUF)]
        + [pltpu.SemaphoreType.DMA for _ in range(2 * _NBUF)]
    ),
)
def _embed_sc(tok_ref, pos_ref, tab_ref, out_ref, idx_v, pos_v, *rest):
    bufs = rest[:_NBUF]
    oidx = rest[_NBUF:2 * _NBUF]
    gsems = rest[2 * _NBUF:3 * _NBUF]
    ssems = rest[3 * _NBUF:4 * _NBUF]

    wid = lax.axis_index("s") * 2 + lax.axis_index("c")
    base = wid * _RPW
    t0 = base // _B

    pltpu.sync_copy(tok_ref.at[pl.ds(base, _RPW)], idx_v)
    pltpu.sync_copy(pos_ref.at[pl.ds(t0 * _D, 4 * _D)], pos_v)

    def fire_gather(k, b):
        pltpu.async_copy(
            tab_ref.at[idx_v.at[pl.ds(k * _CHUNK, _CHUNK)]], bufs[b], gsems[b])

    def wait_gather(k, b):
        pltpu.make_async_copy(
            tab_ref.at[idx_v.at[pl.ds(k * _CHUNK, _CHUNK)]], bufs[b],
            gsems[b]).wait()

    def wait_store(b):
        pltpu.make_async_copy(bufs[b], out_ref.at[oidx[b]], ssems[b]).wait()

    def add_pos(buf, ti):
        def dv_body(dv, carry):
            off = ti * _D + dv * _LANES
            pv = pos_v[pl.ds(off, _LANES)]
            for r in range(_CHUNK):
                buf[r, pl.ds(dv * _LANES, _LANES)] = (
                    buf[r, pl.ds(dv * _LANES, _LANES)] + pv)
            return carry
        lax.fori_loop(0, _DV, dv_body, 0)

    for b in range(_NBUF):
        fire_gather(b, b)

    _LEAD = _NBUF - 3

    def block(o, carry):
        k0 = o * _NBUF
        for b in range(_NBUF):
            k = k0 + b
            g = base + k * _CHUNK
            t = g // _B
            b0 = g % _B
            ti = t - t0
            oidx[b][...] = lax.iota(jnp.int32, _CHUNK) * _T + (b0 * _T + t)
            wait_gather(k, b)
            add_pos(bufs[b], ti)
            pltpu.async_copy(bufs[b], out_ref.at[oidx[b]], ssems[b])

            b4 = (b + _LEAD) % _NBUF

            @pl.when(jnp.logical_and(k >= _NBUF - _LEAD,
                                     k < _NCH - _LEAD))
            def _retire_refill():
                wait_store(b4)
                fire_gather(k + _LEAD, b4)

        return carry

    lax.fori_loop(0, _NBLK, block, 0)

    for b in range(_NBUF):
        wait_store(b)


def kernel(tokens, token_embd, position_embd):
    tokens_t = tokens.astype(jnp.int32).T.reshape(-1)
    pos_flat = jnp.pad(position_embd, ((0, 3), (0, 0))).reshape(-1)
    out = _embed_sc(tokens_t, pos_flat, token_embd)
    return out.reshape(_B, _T, _D)
